# initial kernel scaffold (unmeasured)
import jax
import jax.numpy as jnp
from jax import lax
from jax.experimental import pallas as pl
from jax.experimental.pallas import tpu as pltpu

B = 32
H = 16
D = 128
BS = 32
X_SIZE = 2

PP = 8

NEG_INF = -1e30

_CompilerParams = getattr(pltpu, "CompilerParams", None) or getattr(
    pltpu, "TPUCompilerParams"
)


def _flash_body(q_ref, k_ref, v_ref, w_ref, acc_ref, m_ref, l_ref):
    p = pl.program_id(1)

    @pl.when(p == 0)
    def _():
        acc_ref[...] = jnp.zeros_like(acc_ref)
        m_ref[...] = jnp.full_like(m_ref, NEG_INF)
        l_ref[...] = jnp.zeros_like(l_ref)

    t = PP * BS
    q = q_ref[:, 0, :]
    k = k_ref[:, :, 0, :].reshape(t, D)
    v = v_ref[:, :, 0, :].reshape(t, D)
    w = w_ref[...]

    s = lax.dot_general(
        q, k, (((1,), (1,)), ((), ())), preferred_element_type=jnp.float32
    ) * (D ** -0.5)

    m_old = m_ref[...]
    m_new = jnp.maximum(m_old, jnp.max(s, axis=1, keepdims=True))
    alpha = jnp.exp(m_old - m_new)
    e = w * jnp.exp(s - m_new)
    pv = lax.dot_general(
        e, v, (((1,), (0,)), ((), ())), preferred_element_type=jnp.float32
    )

    m_ref[...] = m_new
    l_ref[...] = alpha * l_ref[...] + jnp.sum(e, axis=1, keepdims=True)
    acc_ref[:, 0, :] = alpha * acc_ref[:, 0, :] + pv


def _merge_body(
    acc_ref, m_ref, l_ref, out_ref, racc, rm, rl, send_sems, recv_sems
):
    my_x = lax.axis_index("x")
    my_y = lax.axis_index("y")
    my_z = lax.axis_index("z")
    partner = (1 - my_x, my_y, my_z)

    bar = pltpu.get_barrier_semaphore()
    pl.semaphore_signal(
        bar, inc=1, device_id=partner, device_id_type=pl.DeviceIdType.MESH
    )
    pl.semaphore_wait(bar, 1)

    copies = []
    for i, (src, dst) in enumerate(
        [(acc_ref, racc), (m_ref, rm), (l_ref, rl)]
    ):
        copies.append(
            pltpu.make_async_remote_copy(
                src_ref=src,
                dst_ref=dst,
                send_sem=send_sems.at[i],
                recv_sem=recv_sems.at[i],
                device_id=partner,
                device_id_type=pl.DeviceIdType.MESH,
            )
        )
    for c in copies:
        c.start()
    for c in copies:
        c.wait()

    m_a = m_ref[...]
    m_b = rm[...]
    m_star = jnp.maximum(m_a, m_b)
    c_a = jnp.exp(m_a - m_star)
    c_b = jnp.exp(m_b - m_star)
    l_star = c_a * l_ref[...] + c_b * rl[...]
    num = c_a[:, :, None] * acc_ref[...] + c_b[:, :, None] * racc[...]
    out_ref[...] = num / l_star[:, :, None]


def kernel(Q, K, V, bt, lens):
    np_local = K.shape[0]
    n_slots = bt.shape[1]
    my_x = lax.axis_index("x")

    base = my_x * np_local
    slot = jnp.arange(n_slots)
    valid = slot[None, :] < lens[:, None]
    local_id = jnp.where(valid, bt, -1) - base
    w = jnp.sum(
        (local_id[:, :, None] == jnp.arange(np_local)[None, None, :]).astype(
            jnp.float32
        ),
        axis=1,
    )
    w_tok = jnp.repeat(w, BS, axis=1)

    q3 = Q[:, 0]

    n_steps = np_local // PP
    acc, m, l = pl.pallas_call(
        _flash_body,
        grid=(H, n_steps),
        in_specs=[
            pl.BlockSpec((B, 1, D), lambda h, p: (0, h, 0)),
            pl.BlockSpec((PP, BS, 1, D), lambda h, p: (p, 0, h, 0)),
            pl.BlockSpec((PP, BS, 1, D), lambda h, p: (p, 0, h, 0)),
            pl.BlockSpec((B, PP * BS), lambda h, p: (0, p)),
        ],
        out_specs=[
            pl.BlockSpec((B, 1, D), lambda h, p: (0, h, 0)),
            pl.BlockSpec((B, 1), lambda h, p: (0, h)),
            pl.BlockSpec((B, 1), lambda h, p: (0, h)),
        ],
        out_shape=[
            jax.ShapeDtypeStruct((B, H, D), jnp.float32),
            jax.ShapeDtypeStruct((B, H), jnp.float32),
            jax.ShapeDtypeStruct((B, H), jnp.float32),
        ],
        compiler_params=_CompilerParams(
            dimension_semantics=("arbitrary", "arbitrary")
        ),
    )(q3, K, V, w_tok)

    out = pl.pallas_call(
        _merge_body,
        out_shape=jax.ShapeDtypeStruct((B, H, D), jnp.float32),
        in_specs=[
            pl.BlockSpec(memory_space=pltpu.VMEM),
            pl.BlockSpec(memory_space=pltpu.VMEM),
            pl.BlockSpec(memory_space=pltpu.VMEM),
        ],
        out_specs=pl.BlockSpec(memory_space=pltpu.VMEM),
        scratch_shapes=[
            pltpu.VMEM((B, H, D), jnp.float32),
            pltpu.VMEM((B, H), jnp.float32),
            pltpu.VMEM((B, H), jnp.float32),
            pltpu.SemaphoreType.DMA((3,)),
            pltpu.SemaphoreType.DMA((3,)),
        ],
        compiler_params=_CompilerParams(collective_id=0),
    )(acc, m, l)

    return out[:, None, :, :]


# baseline (device time: 182938 ns/iter reference)
import jax
import jax.numpy as jnp
from jax import lax
from jax.experimental import pallas as pl
from jax.experimental.pallas import tpu as pltpu

B = 32
H = 16
D = 128
BS = 32
X_SIZE = 2

PP = 8

NEG_INF = -1e30

_CompilerParams = getattr(pltpu, "CompilerParams", None) or getattr(
    pltpu, "TPUCompilerParams"
)


def _flash_body(q_ref, k_ref, v_ref, w_ref, acc_ref, m_ref, l_ref):
    p = pl.program_id(0)

    @pl.when(p == 0)
    def _():
        acc_ref[...] = jnp.zeros_like(acc_ref)
        m_ref[...] = jnp.full_like(m_ref, NEG_INF)
        l_ref[...] = jnp.zeros_like(l_ref)

    t = PP * BS
    w = w_ref[...]
    for h in range(H):
        q = q_ref[:, h, :]
        k = k_ref[:, :, h, :].reshape(t, D)
        v = v_ref[:, :, h, :].reshape(t, D)

        s = lax.dot_general(
            q, k, (((1,), (1,)), ((), ())),
            preferred_element_type=jnp.float32,
        ) * (D ** -0.5)

        m_old = m_ref[:, h : h + 1]
        m_new = jnp.maximum(m_old, jnp.max(s, axis=1, keepdims=True))
        alpha = jnp.exp(m_old - m_new)
        e = w * jnp.exp(s - m_new)
        pv = lax.dot_general(
            e, v, (((1,), (0,)), ((), ())),
            preferred_element_type=jnp.float32,
        )

        m_ref[:, h : h + 1] = m_new
        l_ref[:, h : h + 1] = (
            alpha * l_ref[:, h : h + 1] + jnp.sum(e, axis=1, keepdims=True)
        )
        acc_ref[:, h, :] = alpha * acc_ref[:, h, :] + pv


def _merge_body(
    acc_ref, m_ref, l_ref, out_ref, racc, rm, rl, send_sems, recv_sems
):
    my_x = lax.axis_index("x")
    my_y = lax.axis_index("y")
    my_z = lax.axis_index("z")
    partner = (1 - my_x, my_y, my_z)

    bar = pltpu.get_barrier_semaphore()
    pl.semaphore_signal(
        bar, inc=1, device_id=partner, device_id_type=pl.DeviceIdType.MESH
    )
    pl.semaphore_wait(bar, 1)

    copies = []
    for i, (src, dst) in enumerate(
        [(acc_ref, racc), (m_ref, rm), (l_ref, rl)]
    ):
        copies.append(
            pltpu.make_async_remote_copy(
                src_ref=src,
                dst_ref=dst,
                send_sem=send_sems.at[i],
                recv_sem=recv_sems.at[i],
                device_id=partner,
                device_id_type=pl.DeviceIdType.MESH,
            )
        )
    for c in copies:
        c.start()
    for c in copies:
        c.wait()

    m_a = m_ref[...]
    m_b = rm[...]
    m_star = jnp.maximum(m_a, m_b)
    c_a = jnp.exp(m_a - m_star)
    c_b = jnp.exp(m_b - m_star)
    l_star = c_a * l_ref[...] + c_b * rl[...]
    num = c_a[:, :, None] * acc_ref[...] + c_b[:, :, None] * racc[...]
    out_ref[...] = num / l_star[:, :, None]


def kernel(Q, K, V, bt, lens):
    np_local = K.shape[0]
    n_slots = bt.shape[1]
    my_x = lax.axis_index("x")

    base = my_x * np_local
    slot = jnp.arange(n_slots)
    valid = slot[None, :] < lens[:, None]
    local_id = jnp.where(valid, bt, -1) - base
    w = jnp.sum(
        (local_id[:, :, None] == jnp.arange(np_local)[None, None, :]).astype(
            jnp.float32
        ),
        axis=1,
    )
    w_tok = jnp.repeat(w, BS, axis=1)

    q3 = Q[:, 0]

    n_steps = np_local // PP
    acc, m, l = pl.pallas_call(
        _flash_body,
        grid=(n_steps,),
        in_specs=[
            pl.BlockSpec((B, H, D), lambda p: (0, 0, 0)),
            pl.BlockSpec((PP, BS, H, D), lambda p: (p, 0, 0, 0)),
            pl.BlockSpec((PP, BS, H, D), lambda p: (p, 0, 0, 0)),
            pl.BlockSpec((B, PP * BS), lambda p: (0, p)),
        ],
        out_specs=[
            pl.BlockSpec((B, H, D), lambda p: (0, 0, 0)),
            pl.BlockSpec((B, H), lambda p: (0, 0)),
            pl.BlockSpec((B, H), lambda p: (0, 0)),
        ],
        out_shape=[
            jax.ShapeDtypeStruct((B, H, D), jnp.float32),
            jax.ShapeDtypeStruct((B, H), jnp.float32),
            jax.ShapeDtypeStruct((B, H), jnp.float32),
        ],
        compiler_params=_CompilerParams(
            dimension_semantics=("arbitrary",)
        ),
    )(q3, K, V, w_tok)

    out = pl.pallas_call(
        _merge_body,
        out_shape=jax.ShapeDtypeStruct((B, H, D), jnp.float32),
        in_specs=[
            pl.BlockSpec(memory_space=pltpu.VMEM),
            pl.BlockSpec(memory_space=pltpu.VMEM),
            pl.BlockSpec(memory_space=pltpu.VMEM),
        ],
        out_specs=pl.BlockSpec(memory_space=pltpu.VMEM),
        scratch_shapes=[
            pltpu.VMEM((B, H, D), jnp.float32),
            pltpu.VMEM((B, H), jnp.float32),
            pltpu.VMEM((B, H), jnp.float32),
            pltpu.SemaphoreType.DMA((3,)),
            pltpu.SemaphoreType.DMA((3,)),
        ],
        compiler_params=_CompilerParams(collective_id=0),
    )(acc, m, l)

    return out[:, None, :, :]
